# baseline (speedup 1.0000x reference)
import functools

import jax
import jax.numpy as jnp
from jax import lax
from jax.experimental import pallas as pl
from jax.experimental.pallas import tpu as pltpu
from jax.experimental.pallas import tpu_sc as plsc

N_TOKENS = 16384
DIM = 4096
N_EXPERTS = 64
TOP_K = 8

BLOCK_T = 1024
CHUNK_T = 256

_info = plsc.get_sparse_core_info()
_NC, _NS, _L = _info.num_cores, _info.num_subcores, _info.num_lanes
_NW = _NC * _NS
_TPW = N_TOKENS // _NW


def _softmax_body(x_ref, wt_ref, ori_ref):
    wt = wt_ref[...]
    for r in range(0, BLOCK_T, CHUNK_T):
        x = x_ref[pl.ds(r, CHUNK_T), :]
        score = jax.lax.dot_general(
            x, wt, (((1,), (0,)), ((), ())),
            preferred_element_type=jnp.float32,
        )
        m = jnp.max(score, axis=1, keepdims=True)
        e = jnp.exp(score - m)
        ori_ref[pl.ds(r, CHUNK_T), :] = e / jnp.sum(e, axis=1, keepdims=True)


@jax.jit
def _softmax_scores(x, wt):
    return pl.pallas_call(
        _softmax_body,
        grid=(N_TOKENS // BLOCK_T,),
        in_specs=[
            pl.BlockSpec((BLOCK_T, DIM), lambda i: (i, 0)),
            pl.BlockSpec((DIM, N_EXPERTS), lambda i: (0, 0)),
        ],
        out_specs=pl.BlockSpec((BLOCK_T, N_EXPERTS), lambda i: (i, 0)),
        out_shape=jax.ShapeDtypeStruct((N_TOKENS, N_EXPERTS), jnp.float32),
        compiler_params=pltpu.CompilerParams(
            dimension_semantics=("arbitrary",),
        ),
    )(x, wt)


def _merge_desc(ka, va, kb, vb):
    rkb = lax.rev(kb, (0,))
    rvb = lax.rev(vb, (0,))
    take_a = ka >= rkb
    ku = jnp.where(take_a, ka, rkb)
    vu = jnp.where(take_a, va, rvb)
    return plsc.sort_key_val(ku, vu, descending=True)


_HALF = _TPW // 2


def _sc_topk_kernel(ori_hbm, bias_hbm, out_hbm, buf, stage, bias_v, sem):
    wid = lax.axis_index("s") * _NC + lax.axis_index("c")
    base = wid * _TPW
    pltpu.sync_copy(bias_hbm, bias_v)

    iota = lax.iota(jnp.int32, _L)
    lane_f = iota.astype(jnp.float32)
    segs = [lane_f + jnp.float32(16 * k) for k in range(4)]
    bias_segs = [bias_v[pl.ds(16 * k, 16)] for k in range(4)]
    low = iota < TOP_K

    def token(t, _):
        ks = []
        vs = []
        for k in range(4):
            ori_k = buf[t, pl.ds(16 * k, 16)]
            biased_k = ori_k + bias_segs[k]
            sk, sv = plsc.sort_key_val(biased_k, segs[k], descending=True)
            ks.append(sk)
            vs.append(sv)
        k01, v01 = _merge_desc(ks[0], vs[0], ks[1], vs[1])
        k23, v23 = _merge_desc(ks[2], vs[2], ks[3], vs[3])
        kf, vf = _merge_desc(k01, v01, k23, v23)
        idx_i = vf.astype(jnp.int32)
        bias_g = plsc.load_gather(bias_v, [idx_i])
        w = kf - bias_g
        combined = jnp.where(low, w, lax.rev(vf, (0,)))
        stage[t, :] = combined
        return 0

    for h in range(2):
        hbase = base + h * _HALF
        pltpu.sync_copy(ori_hbm.at[pl.ds(hbase, _HALF)], buf)
        lax.fori_loop(0, _HALF, token, 0)
        pltpu.sync_copy(stage, out_hbm.at[pl.ds(hbase, _HALF)])


@jax.jit
def _sc_topk(ori, bias):
    mesh = plsc.VectorSubcoreMesh(core_axis_name="c", subcore_axis_name="s")
    return pl.kernel(
        _sc_topk_kernel,
        mesh=mesh,
        out_type=jax.ShapeDtypeStruct((N_TOKENS, _L), jnp.float32),
        scratch_types=[
            pltpu.VMEM((_HALF, N_EXPERTS), jnp.float32),
            pltpu.VMEM((_HALF, _L), jnp.float32),
            pltpu.VMEM((N_EXPERTS,), jnp.float32),
            pltpu.SemaphoreType.DMA,
        ],
        compiler_params=pltpu.CompilerParams(needs_layout_passes=False),
    )(ori, bias)


def kernel(x, W, bias):
    ori = _softmax_scores(x, W.T)
    packed = _sc_topk(ori, bias)
    weight = packed[:, :TOP_K]
    idx = packed[:, 2 * TOP_K - 1:TOP_K - 1:-1].astype(jnp.int32)
    return (weight, idx, jnp.float32(0.0))


# speedup vs baseline: 1.7950x; 1.7950x over previous
import jax
import jax.numpy as jnp
from jax.experimental import pallas as pl
from jax.experimental.pallas import tpu as pltpu

N_TOKENS = 16384
DIM = 4096
N_EXPERTS = 64
TOP_K = 8

BLOCK_T = 1024
CHUNK_T = 512


def _topk_rows(score, bias_row):
    m = jnp.max(score, axis=1, keepdims=True)
    e = jnp.exp(score - m)
    ori = e / jnp.sum(e, axis=1, keepdims=True)
    biased = ori + bias_row

    iota_f = jax.lax.broadcasted_iota(jnp.int32, biased.shape, 1).astype(
        jnp.float32)
    neg_inf = jnp.float32(-jnp.inf)
    big = jnp.float32(N_EXPERTS)
    ajs = []
    wjs = []
    for _ in range(TOP_K):
        mj = jnp.max(biased, axis=1, keepdims=True)
        cand = jnp.where(biased == mj, iota_f, big)
        aj = jnp.min(cand, axis=1, keepdims=True)
        onehot = iota_f == aj
        wj = jnp.sum(jnp.where(onehot, ori, 0.0), axis=1, keepdims=True)
        ajs.append(aj)
        wjs.append(wj)
        biased = jnp.where(onehot, neg_inf, biased)
    weight = jnp.concatenate(wjs, axis=1)
    idx = jnp.concatenate(ajs, axis=1).astype(jnp.int32)
    return weight, idx


def _router_body(x_ref, wt_ref, bias_ref, weight_ref, idx_ref):
    wt = wt_ref[...]
    bias_row = bias_ref[...]
    for r in range(0, BLOCK_T, CHUNK_T):
        x = x_ref[pl.ds(r, CHUNK_T), :]
        score = jax.lax.dot_general(
            x, wt, (((1,), (0,)), ((), ())),
            preferred_element_type=jnp.float32,
        )
        weight, idx = _topk_rows(score, bias_row)
        weight_ref[pl.ds(r, CHUNK_T), :] = weight
        idx_ref[pl.ds(r, CHUNK_T), :] = idx


@jax.jit
def _router(x, wt, bias):
    grid = (N_TOKENS // BLOCK_T,)
    return pl.pallas_call(
        _router_body,
        grid=grid,
        in_specs=[
            pl.BlockSpec((BLOCK_T, DIM), lambda i: (i, 0)),
            pl.BlockSpec((DIM, N_EXPERTS), lambda i: (0, 0)),
            pl.BlockSpec((1, N_EXPERTS), lambda i: (0, 0)),
        ],
        out_specs=[
            pl.BlockSpec((BLOCK_T, TOP_K), lambda i: (i, 0)),
            pl.BlockSpec((BLOCK_T, TOP_K), lambda i: (i, 0)),
        ],
        out_shape=[
            jax.ShapeDtypeStruct((N_TOKENS, TOP_K), jnp.float32),
            jax.ShapeDtypeStruct((N_TOKENS, TOP_K), jnp.int32),
        ],
        compiler_params=pltpu.CompilerParams(
            dimension_semantics=("arbitrary",),
        ),
    )(x, wt, bias)


def kernel(x, W, bias):
    weight, idx = _router(x, W.T, bias.reshape(1, N_EXPERTS))
    return (weight, idx, jnp.float32(0.0))


# speedup vs baseline: 1.9007x; 1.0589x over previous
import jax
import jax.numpy as jnp
from jax.experimental import pallas as pl
from jax.experimental.pallas import tpu as pltpu

N_TOKENS = 16384
DIM = 4096
N_EXPERTS = 64
TOP_K = 8

BLOCK_T = 1024
CHUNK_T = 256


def _topk_rows(score, bias_row):
    m = jnp.max(score, axis=1, keepdims=True)
    e = jnp.exp(score - m)
    ori = e / jnp.sum(e, axis=1, keepdims=True)
    biased = ori + bias_row

    iota_f = jax.lax.broadcasted_iota(jnp.int32, biased.shape, 1).astype(
        jnp.float32)
    neg_inf = jnp.float32(-jnp.inf)
    big = jnp.float32(N_EXPERTS)
    ajs = []
    wjs = []
    for _ in range(TOP_K):
        mj = jnp.max(biased, axis=1, keepdims=True)
        cand = jnp.where(biased == mj, iota_f, big)
        aj = jnp.min(cand, axis=1, keepdims=True)
        onehot = iota_f == aj
        wj = jnp.sum(jnp.where(onehot, ori, 0.0), axis=1, keepdims=True)
        ajs.append(aj)
        wjs.append(wj)
        biased = jnp.where(onehot, neg_inf, biased)
    weight = jnp.concatenate(wjs, axis=1)
    idx = jnp.concatenate(ajs, axis=1).astype(jnp.int32)
    return weight, idx


def _router_body(xa_ref, xb_ref, wt_ref, bias_ref, weight_ref, idx_ref):
    wt = wt_ref[...]
    bias_row = bias_ref[...]
    half = DIM // 2
    for r in range(0, BLOCK_T, CHUNK_T):
        xa = xa_ref[pl.ds(r, CHUNK_T), :]
        xb = xb_ref[pl.ds(r, CHUNK_T), :]
        score = jax.lax.dot_general(
            xa, wt[:half], (((1,), (0,)), ((), ())),
            preferred_element_type=jnp.float32,
        ) + jax.lax.dot_general(
            xb, wt[half:], (((1,), (0,)), ((), ())),
            preferred_element_type=jnp.float32,
        )
        weight, idx = _topk_rows(score, bias_row)
        weight_ref[pl.ds(r, CHUNK_T), :] = weight
        idx_ref[pl.ds(r, CHUNK_T), :] = idx


@jax.jit
def _router(x, wt, bias):
    grid = (N_TOKENS // BLOCK_T,)
    return pl.pallas_call(
        _router_body,
        grid=grid,
        in_specs=[
            pl.BlockSpec((BLOCK_T, DIM // 2), lambda i: (i, 0)),
            pl.BlockSpec((BLOCK_T, DIM // 2), lambda i: (i, 1)),
            pl.BlockSpec((DIM, N_EXPERTS), lambda i: (0, 0)),
            pl.BlockSpec((1, N_EXPERTS), lambda i: (0, 0)),
        ],
        out_specs=[
            pl.BlockSpec((BLOCK_T, TOP_K), lambda i: (i, 0)),
            pl.BlockSpec((BLOCK_T, TOP_K), lambda i: (i, 0)),
        ],
        out_shape=[
            jax.ShapeDtypeStruct((N_TOKENS, TOP_K), jnp.float32),
            jax.ShapeDtypeStruct((N_TOKENS, TOP_K), jnp.int32),
        ],
        compiler_params=pltpu.CompilerParams(
            dimension_semantics=("arbitrary",),
            vmem_limit_bytes=100 * 1024 * 1024,
        ),
    )(x, x, wt, bias)


def kernel(x, W, bias):
    weight, idx = _router(x, W.T, bias.reshape(1, N_EXPERTS))
    return (weight, idx, jnp.float32(0.0))
